# baseline (speedup 1.0000x reference)
import functools

import jax
import jax.numpy as jnp
from jax import lax
from jax.experimental import pallas as pl
from jax.experimental.pallas import tpu as pltpu
from jax.experimental.pallas import tpu_sc as plsc

N = 320000
D = 128
H = 128
B = 1024

NW = 32
CROWS = N // NW
T = 400
NT = CROWS // T

MLP_R = 2048


def _mlp_body(x_ref, posp_ref, w1x_ref, w1p_ref, b1_ref, w2_ref, b2_ref, o_ref):
    xb = x_ref[...].astype(jnp.bfloat16)
    pb = posp_ref[...].astype(jnp.bfloat16)
    h = jnp.dot(xb, w1x_ref[...], preferred_element_type=jnp.float32)
    h += jnp.dot(pb, w1p_ref[...], preferred_element_type=jnp.float32)
    h = jnp.maximum(h + b1_ref[...], 0.0).astype(jnp.bfloat16)
    h = jnp.dot(h, w2_ref[...], preferred_element_type=jnp.float32)
    o_ref[...] = h + b2_ref[...]


def _mlp(x, posp, w1x, w1p, b1, w2, b2):
    grid = (N // MLP_R,)
    return pl.pallas_call(
        _mlp_body,
        grid=grid,
        in_specs=[
            pl.BlockSpec((MLP_R, D), lambda k: (k, 0)),
            pl.BlockSpec((MLP_R, 8), lambda k: (k, 0)),
            pl.BlockSpec((D, H), lambda k: (0, 0)),
            pl.BlockSpec((8, H), lambda k: (0, 0)),
            pl.BlockSpec((1, H), lambda k: (0, 0)),
            pl.BlockSpec((H, H), lambda k: (0, 0)),
            pl.BlockSpec((1, H), lambda k: (0, 0)),
        ],
        out_specs=pl.BlockSpec((MLP_R, H), lambda k: (k, 0)),
        out_shape=jax.ShapeDtypeStruct((N, H), jnp.float32),
    )(x, posp, w1x, w1p, b1, w2, b2)


_NEG = float(jnp.finfo(jnp.float32).min)


def _seg_phase1_body(h_hbm, ids_hbm, o1_hbm, side_hbm, data_v, ids_v, stage_v, sem):
    cid = lax.axis_index("c")
    sid = lax.axis_index("s")
    wid = sid * 2 + cid
    base = wid * CROWS

    def flush(prev, fid, m):
        for k in range(8):
            stage_v[pl.ds(k * 16, 16)] = m[k]

        def to_side():
            pltpu.sync_copy(stage_v, side_hbm.at[wid])

        def to_out():
            pltpu.sync_copy(stage_v, o1_hbm.at[prev])

        lax.cond(prev == fid, to_side, to_out)

    def tile_loop(t, carry):
        r0 = base + t * T
        pltpu.sync_copy(h_hbm.at[pl.ds(r0, T)], data_v)
        pltpu.sync_copy(ids_hbm.at[pl.ds(r0, T)], ids_v)

        def group_loop(q, gcarry):
            prev, fid = gcarry[0], gcarry[1]
            m = list(gcarry[2:])
            ids16 = ids_v[pl.ds(q * 16, 16)]
            for j in range(16):
                r = q * 16 + j
                i = ids16[j]
                d = [data_v[r, pl.ds(k * 16, 16)] for k in range(8)]
                first = fid < 0
                fid = jnp.where(first, i, fid)
                prev = jnp.where(first, i, prev)
                changed = i != prev

                def on_change(prev=prev, fid=fid, m=m):
                    flush(prev, fid, m)

                lax.cond(changed, on_change, lambda: None)
                neg = jnp.full((16,), _NEG, jnp.float32)
                m = [jnp.maximum(jnp.where(changed, neg, m[k]), d[k])
                     for k in range(8)]
                prev = i
            return (prev, fid) + tuple(m)

        return lax.fori_loop(0, T // 16, group_loop, carry)

    init = (jnp.int32(-1), jnp.int32(-1)) + tuple(
        jnp.full((16,), _NEG, jnp.float32) for _ in range(8)
    )
    final = lax.fori_loop(0, NT, tile_loop, init)
    prev, fid = final[0], final[1]
    m = list(final[2:])
    flush(prev, fid, m)


def _seg_phase1(h, ids):
    mesh = plsc.VectorSubcoreMesh(core_axis_name="c", subcore_axis_name="s")
    f = pl.kernel(
        _seg_phase1_body,
        out_type=[
            jax.ShapeDtypeStruct((B, H), jnp.float32),
            jax.ShapeDtypeStruct((NW, H), jnp.float32),
        ],
        mesh=mesh,
        scratch_types=[
            pltpu.VMEM((T, H), jnp.float32),
            pltpu.VMEM((T,), jnp.int32),
            pltpu.VMEM((H,), jnp.float32),
            pltpu.SemaphoreType.DMA,
        ],
    )
    return f(h, ids)


RPW = B // NW


def _seg_phase2_body(o1_hbm, side_hbm, code_hbm, wlo_hbm, whi_hbm, o2_hbm,
                     o1_v, side_v, code_v, wlo_v, whi_v, out_v, sem):
    cid = lax.axis_index("c")
    sid = lax.axis_index("s")
    wid = sid * 2 + cid
    base = wid * RPW

    pltpu.sync_copy(o1_hbm.at[pl.ds(base, RPW)], o1_v)
    pltpu.sync_copy(side_hbm, side_v)
    pltpu.sync_copy(code_hbm.at[pl.ds(base, RPW)], code_v)
    pltpu.sync_copy(wlo_hbm.at[pl.ds(base, RPW)], wlo_v)
    pltpu.sync_copy(whi_hbm.at[pl.ds(base, RPW)], whi_v)

    for q in range(RPW // 16):
        code16 = code_v[pl.ds(q * 16, 16)]
        wlo16 = wlo_v[pl.ds(q * 16, 16)]
        whi16 = whi_v[pl.ds(q * 16, 16)]
        for j in range(16):
            row = q * 16 + j
            c = code16[j]
            lo = wlo16[j]
            hi = whi16[j]
            val = []
            for k in range(8):
                o1k = o1_v[row, pl.ds(k * 16, 16)]
                v = jnp.where(c == 1, o1k,
                              jnp.where(c == 0, jnp.zeros((16,), jnp.float32),
                                        jnp.full((16,), _NEG, jnp.float32)))
                val.append(v)

            def side_loop(w, vcarry):
                return tuple(
                    jnp.maximum(vcarry[k], side_v[w, pl.ds(k * 16, 16)])
                    for k in range(8)
                )

            val = lax.fori_loop(lo, hi, side_loop, tuple(val))
            for k in range(8):
                out_v[row, pl.ds(k * 16, 16)] = val[k]

    pltpu.sync_copy(out_v, o2_hbm.at[pl.ds(base, RPW)])


def _seg_phase2(o1, side, code, wlo, whi):
    mesh = plsc.VectorSubcoreMesh(core_axis_name="c", subcore_axis_name="s")
    f = pl.kernel(
        _seg_phase2_body,
        out_type=jax.ShapeDtypeStruct((B, H), jnp.float32),
        mesh=mesh,
        scratch_types=[
            pltpu.VMEM((RPW, H), jnp.float32),
            pltpu.VMEM((NW, H), jnp.float32),
            pltpu.VMEM((RPW,), jnp.int32),
            pltpu.VMEM((RPW,), jnp.int32),
            pltpu.VMEM((RPW,), jnp.int32),
            pltpu.VMEM((RPW, H), jnp.float32),
            pltpu.SemaphoreType.DMA,
        ],
    )
    return f(o1, side, code, wlo, whi)


@jax.jit
def _run(x, pos, batch, W1, b1, W2, b2):
    posp = jnp.zeros((N, 8), jnp.float32).at[:, :3].set(pos)
    w1x = W1[:D].astype(jnp.bfloat16)
    w1p = jnp.zeros((8, H), jnp.float32).at[:3].set(W1[D:]).astype(jnp.bfloat16)
    b1r = b1.reshape(1, H)
    b2r = b2.reshape(1, H)

    h = _mlp(x, posp, w1x, w1p, b1r, w2=W2.astype(jnp.bfloat16), b2=b2r)

    ids = batch.astype(jnp.int32)
    cw = jnp.arange(NW, dtype=jnp.int32) * CROWS
    fids = ids[cw]
    lids = ids[cw + CROWS - 1]
    s = jnp.arange(B, dtype=jnp.int32)
    directly = jnp.any((fids[None, :] < s[:, None]) & (s[:, None] <= lids[None, :]),
                       axis=1)
    wlo = jnp.searchsorted(fids, s, side="left").astype(jnp.int32)
    whi = jnp.searchsorted(fids, s, side="right").astype(jnp.int32)
    nonempty = directly | (wlo < whi)
    code = jnp.where(nonempty, jnp.where(directly, 1, 2), 0).astype(jnp.int32)

    o1, side = _seg_phase1(h, ids)
    out = _seg_phase2(o1, side, code, wlo, whi)

    pos_out = jnp.zeros((B, 3), dtype=pos.dtype)
    batch_out = jnp.arange(B, dtype=batch.dtype)
    return (out, pos_out, batch_out)


def kernel(x, pos, batch, W1, b1, W2, b2):
    return _run(x, pos, batch, W1, b1, W2, b2)


# speedup vs baseline: 1.2167x; 1.2167x over previous
import functools

import jax
import jax.numpy as jnp
import numpy as np
from jax import lax
from jax.experimental import pallas as pl
from jax.experimental.pallas import tpu as pltpu
from jax.experimental.pallas import tpu_sc as plsc

N = 320000
D = 128
H = 128
B = 1024

NW = 32
CROWS = N // NW
T = 400
NT = CROWS // T
KC = 8

MLP_R = 2000

_NEG = float(jnp.finfo(jnp.bfloat16).min)


def _mlp_body(x_ref, pos_ref, w1x_ref, w1p_ref, b1_ref, w2_ref, b2_ref, o_ref):
    xb = x_ref[...].astype(jnp.bfloat16)
    pb = pos_ref[...].astype(jnp.bfloat16)
    h = jnp.dot(xb, w1x_ref[...], preferred_element_type=jnp.float32)
    h += jnp.dot(pb, w1p_ref[...], preferred_element_type=jnp.float32)
    h = jnp.maximum(h + b1_ref[...], 0.0).astype(jnp.bfloat16)
    h = jnp.dot(h, w2_ref[...], preferred_element_type=jnp.float32)
    o_ref[...] = (h + b2_ref[...]).astype(jnp.bfloat16)


def _mlp(x, pos, w1x, w1p, b1, w2, b2):
    grid = (N // MLP_R,)
    return pl.pallas_call(
        _mlp_body,
        grid=grid,
        in_specs=[
            pl.BlockSpec((MLP_R, D), lambda k: (k, 0)),
            pl.BlockSpec((MLP_R, 3), lambda k: (k, 0)),
            pl.BlockSpec((D, H), lambda k: (0, 0)),
            pl.BlockSpec((3, H), lambda k: (0, 0)),
            pl.BlockSpec((1, H), lambda k: (0, 0)),
            pl.BlockSpec((H, H), lambda k: (0, 0)),
            pl.BlockSpec((1, H), lambda k: (0, 0)),
        ],
        out_specs=pl.BlockSpec((MLP_R, H), lambda k: (k, 0)),
        out_shape=jax.ShapeDtypeStruct((N, H), jnp.bfloat16),
    )(x, pos, w1x, w1p, b1, w2, b2)


def _seg_phase1_body(h_hbm, ids_hbm, bias_hbm, o1_hbm, side_hbm,
                     data_v, ids_v, m_buf, bias_v, sem):
    cid = lax.axis_index("c")
    sid = lax.axis_index("s")
    wid = sid * 2 + cid
    base = wid * CROWS

    neg2 = jnp.full((2, 16), _NEG, jnp.bfloat16)
    pltpu.sync_copy(bias_hbm, bias_v)

    def flush(pid, fid):
        def to_side():
            pltpu.sync_copy(m_buf, side_hbm.at[pl.ds(wid * 8, 8)])

        def to_out():
            pltpu.sync_copy(m_buf, o1_hbm.at[pl.ds(pid * 8, 8)])

        lax.cond(pid == fid, to_side, to_out)

    def tile_loop(t, carry):
        r0 = base + t * T
        pltpu.sync_copy(h_hbm.at[pl.ds(r0, T)], data_v)
        pltpu.sync_copy(ids_hbm.at[pl.ds(r0, T)], ids_v)

        def group_loop(q, gcarry):
            prev, fid = gcarry
            rbase = pl.multiple_of(q * 16, 16)
            ids16 = ids_v[pl.ds(rbase, 16)]
            i_first = ids16[0]
            i_last = ids16[15]
            uniform = (i_first == prev) & (i_last == prev)

            def fast_group():
                for c in range(KC):
                    acc = m_buf[pl.ds(0, 2), pl.ds(c * 16, 16)]
                    for u in range(8):
                        d = data_v[pl.ds(rbase + 2 * u, 2), pl.ds(c * 16, 16)]
                        acc = jnp.maximum(acc, d)
                    m_buf[pl.ds(0, 2), pl.ds(c * 16, 16)] = acc

            def slow_group():
                prev2, fid2 = prev, fid
                for u in range(8):
                    i0 = ids16[2 * u]
                    i1 = ids16[2 * u + 1]
                    first = fid2 < 0
                    fid2 = jnp.where(first, i0, fid2)
                    prev2 = jnp.where(first, i0, prev2)
                    flush0 = i0 != prev2
                    flush1 = i1 != i0

                    def do_flush0(pid=prev2, f=fid2):
                        flush(pid, f)

                    lax.cond(flush0, do_flush0, lambda: None)
                    mids = []
                    for c in range(KC):
                        d = data_v[pl.ds(rbase + 2 * u, 2), pl.ds(c * 16, 16)]
                        mold = m_buf[pl.ds(0, 2), pl.ds(c * 16, 16)]
                        b0 = bias_v[pl.ds(0, 2), pl.ds(c * 16, 16)]
                        m_mid = jnp.maximum(jnp.where(flush0, neg2, mold),
                                            d + b0)
                        m_buf[pl.ds(0, 2), pl.ds(c * 16, 16)] = m_mid
                        mids.append((m_mid, d))

                    def do_flush1(pid=i0, f=fid2):
                        flush(pid, f)

                    lax.cond(flush1, do_flush1, lambda: None)
                    for c in range(KC):
                        m_mid, d = mids[c]
                        b1 = bias_v[pl.ds(2, 2), pl.ds(c * 16, 16)]
                        m_new = jnp.maximum(jnp.where(flush1, neg2, m_mid),
                                            d + b1)
                        m_buf[pl.ds(0, 2), pl.ds(c * 16, 16)] = m_new
                    prev2 = i1

            lax.cond(uniform, fast_group, slow_group)
            new_fid = jnp.where(fid < 0, i_first, fid)
            return (i_last, new_fid)

        return lax.fori_loop(0, T // 16, group_loop, carry)

    for c in range(KC):
        m_buf[pl.ds(0, 2), pl.ds(c * 16, 16)] = neg2
    prev, fid = lax.fori_loop(0, NT, tile_loop,
                              (jnp.int32(-1), jnp.int32(-1)))
    flush(prev, fid)


def _seg_phase1(h, ids, bias):
    mesh = plsc.VectorSubcoreMesh(core_axis_name="c", subcore_axis_name="s")
    f = pl.kernel(
        _seg_phase1_body,
        out_type=[
            jax.ShapeDtypeStruct((B * 8, H), jnp.bfloat16),
            jax.ShapeDtypeStruct((NW * 8, H), jnp.bfloat16),
        ],
        mesh=mesh,
        scratch_types=[
            pltpu.VMEM((T, H), jnp.bfloat16),
            pltpu.VMEM((T,), jnp.int32),
            pltpu.VMEM((8, H), jnp.bfloat16),
            pltpu.VMEM((8, H), jnp.bfloat16),
            pltpu.SemaphoreType.DMA,
        ],
    )
    return f(h, ids, bias)


RPW = B // NW


def _seg_phase2_body(o1_hbm, side_hbm, code_hbm, wlo_hbm, whi_hbm, o2_hbm,
                     o1_v, side_v, code_v, wlo_v, whi_v, out_v, sem):
    cid = lax.axis_index("c")
    sid = lax.axis_index("s")
    wid = sid * 2 + cid
    base = wid * RPW

    neg2 = jnp.full((2, 16), _NEG, jnp.bfloat16)
    zero2 = jnp.zeros((2, 16), jnp.bfloat16)

    pltpu.sync_copy(o1_hbm.at[pl.ds(base * 8, RPW * 8)], o1_v)
    pltpu.sync_copy(side_hbm, side_v)
    pltpu.sync_copy(code_hbm.at[pl.ds(base, RPW)], code_v)
    pltpu.sync_copy(wlo_hbm.at[pl.ds(base, RPW)], wlo_v)
    pltpu.sync_copy(whi_hbm.at[pl.ds(base, RPW)], whi_v)

    for q in range(RPW // 16):
        code16 = code_v[pl.ds(q * 16, 16)]
        wlo16 = wlo_v[pl.ds(q * 16, 16)]
        whi16 = whi_v[pl.ds(q * 16, 16)]
        for j in range(16):
            row = q * 16 + j
            c = code16[j]
            lo = wlo16[j]
            hi = whi16[j]
            val = []
            for k in range(KC):
                o1k = o1_v[pl.ds(row * 8, 2), pl.ds(k * 16, 16)]
                v = jnp.where(c == 1, o1k, jnp.where(c == 0, zero2, neg2))
                val.append(v)

            def side_loop(w, vcarry):
                w8 = pl.multiple_of(w * 8, 8)
                return tuple(
                    jnp.maximum(vcarry[k],
                                side_v[pl.ds(w8, 2), pl.ds(k * 16, 16)])
                    for k in range(KC)
                )

            val = lax.fori_loop(lo, hi, side_loop, tuple(val))
            for k in range(KC):
                out_v[pl.ds(row * 2, 2), pl.ds(k * 16, 16)] = val[k]

    pltpu.sync_copy(out_v, o2_hbm.at[pl.ds(base * 2, RPW * 2)])


def _seg_phase2(o1, side, code, wlo, whi):
    mesh = plsc.VectorSubcoreMesh(core_axis_name="c", subcore_axis_name="s")
    f = pl.kernel(
        _seg_phase2_body,
        out_type=jax.ShapeDtypeStruct((B * 2, H), jnp.bfloat16),
        mesh=mesh,
        scratch_types=[
            pltpu.VMEM((RPW * 8, H), jnp.bfloat16),
            pltpu.VMEM((NW * 8, H), jnp.bfloat16),
            pltpu.VMEM((RPW,), jnp.int32),
            pltpu.VMEM((RPW,), jnp.int32),
            pltpu.VMEM((RPW,), jnp.int32),
            pltpu.VMEM((RPW * 2, H), jnp.bfloat16),
            pltpu.SemaphoreType.DMA,
        ],
    )
    return f(o1, side, code, wlo, whi)


def _final_body(o2_ref, out_ref):
    x = o2_ref[...].astype(jnp.float32)
    x = x.reshape(B, 2, H)
    out_ref[...] = jnp.max(x, axis=1)


def _finalize(o2):
    return pl.pallas_call(
        _final_body,
        out_shape=jax.ShapeDtypeStruct((B, H), jnp.float32),
    )(o2)


@jax.jit
def _run(x, pos, batch, W1, b1, W2, b2):
    w1x = W1[:D].astype(jnp.bfloat16)
    w1p = W1[D:].astype(jnp.bfloat16)
    b1r = b1.reshape(1, H)
    b2r = b2.reshape(1, H)
    w2 = W2.astype(jnp.bfloat16)

    h = _mlp(x, pos, w1x, w1p, b1r, w2, b2r)

    ids = batch.astype(jnp.int32)
    cw = jnp.arange(NW, dtype=jnp.int32) * CROWS
    fids = ids[cw]
    lids = ids[cw + CROWS - 1]
    s = jnp.arange(B, dtype=jnp.int32)
    directly = jnp.any((fids[None, :] < s[:, None]) & (s[:, None] <= lids[None, :]),
                       axis=1)
    wlo = jnp.searchsorted(fids, s, side="left").astype(jnp.int32)
    whi = jnp.searchsorted(fids, s, side="right").astype(jnp.int32)
    nonempty = directly | (wlo < whi)
    code = jnp.where(nonempty, jnp.where(directly, 1, 2), 0).astype(jnp.int32)

    bias = jnp.zeros((8, H), jnp.bfloat16)
    bias = bias.at[1, :].set(jnp.bfloat16(_NEG)).at[2, :].set(jnp.bfloat16(_NEG))

    o1, side = _seg_phase1(h, ids, bias)
    o2 = _seg_phase2(o1, side, code, wlo, whi)
    out = _finalize(o2)

    pos_out = jnp.zeros((B, 3), dtype=pos.dtype)
    batch_out = jnp.arange(B, dtype=batch.dtype)
    return (out, pos_out, batch_out)


def kernel(x, pos, batch, W1, b1, W2, b2):
    return _run(x, pos, batch, W1, b1, W2, b2)


# speedup vs baseline: 1.3895x; 1.1420x over previous
import functools

import jax
import jax.numpy as jnp
import numpy as np
from jax import lax
from jax.experimental import pallas as pl
from jax.experimental.pallas import tpu as pltpu
from jax.experimental.pallas import tpu_sc as plsc

N = 320000
D = 128
H = 128
B = 1024

NW = 32
CROWS = N // NW
T = 400
NT = CROWS // T
KC = 8

MLP_R = 8000

_NEG = float(jnp.finfo(jnp.bfloat16).min)


def _mlp_body(x_ref, pos_ref, w1x_ref, w1p_ref, b1_ref, w2_ref, b2_ref, o_ref):
    xb = x_ref[...].astype(jnp.bfloat16)
    pb = pos_ref[...].astype(jnp.bfloat16)
    h = jnp.dot(xb, w1x_ref[...], preferred_element_type=jnp.float32)
    h += jnp.dot(pb, w1p_ref[...], preferred_element_type=jnp.float32)
    h = jnp.maximum(h + b1_ref[...], 0.0).astype(jnp.bfloat16)
    h = jnp.dot(h, w2_ref[...], preferred_element_type=jnp.float32)
    o_ref[...] = (h + b2_ref[...]).astype(jnp.bfloat16)


def _mlp(x, pos, w1x, w1p, b1, w2, b2):
    grid = (N // MLP_R,)
    return pl.pallas_call(
        _mlp_body,
        grid=grid,
        in_specs=[
            pl.BlockSpec((MLP_R, D), lambda k: (k, 0)),
            pl.BlockSpec((MLP_R, 3), lambda k: (k, 0)),
            pl.BlockSpec((D, H), lambda k: (0, 0)),
            pl.BlockSpec((3, H), lambda k: (0, 0)),
            pl.BlockSpec((1, H), lambda k: (0, 0)),
            pl.BlockSpec((H, H), lambda k: (0, 0)),
            pl.BlockSpec((1, H), lambda k: (0, 0)),
        ],
        out_specs=pl.BlockSpec((MLP_R, H), lambda k: (k, 0)),
        out_shape=jax.ShapeDtypeStruct((N, H), jnp.bfloat16),
    )(x, pos, w1x, w1p, b1, w2, b2)


def _seg_phase1_body(h_hbm, ids_hbm, bias_hbm, o1_hbm, side_hbm,
                     data_v, ids_v, m_buf, bias_v, sem):
    cid = lax.axis_index("c")
    sid = lax.axis_index("s")
    wid = sid * 2 + cid
    base = wid * CROWS

    neg2 = jnp.full((2, 16), _NEG, jnp.bfloat16)
    pltpu.sync_copy(bias_hbm, bias_v)

    def flush(pid, fid):
        def to_side():
            pltpu.sync_copy(m_buf, side_hbm.at[pl.ds(wid * 8, 8)])

        def to_out():
            pltpu.sync_copy(m_buf, o1_hbm.at[pl.ds(pid * 8, 8)])

        lax.cond(pid == fid, to_side, to_out)

    def tile_loop(t, carry):
        r0 = base + t * T
        pltpu.sync_copy(h_hbm.at[pl.ds(r0, T)], data_v)
        pltpu.sync_copy(ids_hbm.at[pl.ds(r0, T)], ids_v)

        def group_loop(q, gcarry):
            prev, fid = gcarry
            rbase = pl.multiple_of(q * 16, 16)
            ids16 = ids_v[pl.ds(rbase, 16)]
            i_first = ids16[0]
            i_last = ids16[15]
            uniform = (i_first == prev) & (i_last == prev)

            def fast_group():
                for c in range(KC):
                    acc = m_buf[pl.ds(0, 2), pl.ds(c * 16, 16)]
                    for u in range(8):
                        d = data_v[pl.ds(rbase + 2 * u, 2), pl.ds(c * 16, 16)]
                        acc = jnp.maximum(acc, d)
                    m_buf[pl.ds(0, 2), pl.ds(c * 16, 16)] = acc

            def slow_group():
                prev2, fid2 = prev, fid
                for u in range(8):
                    i0 = ids16[2 * u]
                    i1 = ids16[2 * u + 1]
                    first = fid2 < 0
                    fid2 = jnp.where(first, i0, fid2)
                    prev2 = jnp.where(first, i0, prev2)
                    flush0 = i0 != prev2
                    flush1 = i1 != i0

                    def do_flush0(pid=prev2, f=fid2):
                        flush(pid, f)

                    lax.cond(flush0, do_flush0, lambda: None)
                    mids = []
                    for c in range(KC):
                        d = data_v[pl.ds(rbase + 2 * u, 2), pl.ds(c * 16, 16)]
                        mold = m_buf[pl.ds(0, 2), pl.ds(c * 16, 16)]
                        b0 = bias_v[pl.ds(0, 2), pl.ds(c * 16, 16)]
                        m_mid = jnp.maximum(jnp.where(flush0, neg2, mold),
                                            d + b0)
                        m_buf[pl.ds(0, 2), pl.ds(c * 16, 16)] = m_mid
                        mids.append((m_mid, d))

                    def do_flush1(pid=i0, f=fid2):
                        flush(pid, f)

                    lax.cond(flush1, do_flush1, lambda: None)
                    for c in range(KC):
                        m_mid, d = mids[c]
                        b1 = bias_v[pl.ds(2, 2), pl.ds(c * 16, 16)]
                        m_new = jnp.maximum(jnp.where(flush1, neg2, m_mid),
                                            d + b1)
                        m_buf[pl.ds(0, 2), pl.ds(c * 16, 16)] = m_new
                    prev2 = i1

            lax.cond(uniform, fast_group, slow_group)
            new_fid = jnp.where(fid < 0, i_first, fid)
            return (i_last, new_fid)

        return lax.fori_loop(0, T // 16, group_loop, carry)

    for c in range(KC):
        m_buf[pl.ds(0, 2), pl.ds(c * 16, 16)] = neg2
    prev, fid = lax.fori_loop(0, NT, tile_loop,
                              (jnp.int32(-1), jnp.int32(-1)))
    flush(prev, fid)


def _seg_phase1(h, ids, bias):
    mesh = plsc.VectorSubcoreMesh(core_axis_name="c", subcore_axis_name="s")
    f = pl.kernel(
        _seg_phase1_body,
        out_type=[
            jax.ShapeDtypeStruct((B * 8, H), jnp.bfloat16),
            jax.ShapeDtypeStruct((NW * 8, H), jnp.bfloat16),
        ],
        mesh=mesh,
        scratch_types=[
            pltpu.VMEM((T, H), jnp.bfloat16),
            pltpu.VMEM((T,), jnp.int32),
            pltpu.VMEM((8, H), jnp.bfloat16),
            pltpu.VMEM((8, H), jnp.bfloat16),
            pltpu.SemaphoreType.DMA,
        ],
    )
    return f(h, ids, bias)


RPW = B // NW


def _seg_phase2_body(o1_hbm, side_hbm, code_hbm, wlo_hbm, whi_hbm, o2_hbm,
                     o1_v, side_v, code_v, wlo_v, whi_v, out_v, sem):
    cid = lax.axis_index("c")
    sid = lax.axis_index("s")
    wid = sid * 2 + cid
    base = wid * RPW

    neg2 = jnp.full((2, 16), _NEG, jnp.bfloat16)
    zero2 = jnp.zeros((2, 16), jnp.bfloat16)

    pltpu.sync_copy(o1_hbm.at[pl.ds(base * 8, RPW * 8)], o1_v)
    pltpu.sync_copy(side_hbm, side_v)
    pltpu.sync_copy(code_hbm.at[pl.ds(base, RPW)], code_v)
    pltpu.sync_copy(wlo_hbm.at[pl.ds(base, RPW)], wlo_v)
    pltpu.sync_copy(whi_hbm.at[pl.ds(base, RPW)], whi_v)

    for q in range(RPW // 16):
        code16 = code_v[pl.ds(q * 16, 16)]
        wlo16 = wlo_v[pl.ds(q * 16, 16)]
        whi16 = whi_v[pl.ds(q * 16, 16)]
        for j in range(16):
            row = q * 16 + j
            c = code16[j]
            lo = wlo16[j]
            hi = whi16[j]
            val = []
            for k in range(KC):
                o1k = o1_v[pl.ds(row * 8, 2), pl.ds(k * 16, 16)]
                v = jnp.where(c == 1, o1k, jnp.where(c == 0, zero2, neg2))
                val.append(v)

            def side_loop(w, vcarry):
                w8 = pl.multiple_of(w * 8, 8)
                return tuple(
                    jnp.maximum(vcarry[k],
                                side_v[pl.ds(w8, 2), pl.ds(k * 16, 16)])
                    for k in range(KC)
                )

            val = lax.fori_loop(lo, hi, side_loop, tuple(val))
            for k in range(KC):
                out_v[pl.ds(row * 2, 2), pl.ds(k * 16, 16)] = val[k]

    pltpu.sync_copy(out_v, o2_hbm.at[pl.ds(base * 2, RPW * 2)])


def _seg_phase2(o1, side, code, wlo, whi):
    mesh = plsc.VectorSubcoreMesh(core_axis_name="c", subcore_axis_name="s")
    f = pl.kernel(
        _seg_phase2_body,
        out_type=jax.ShapeDtypeStruct((B * 2, H), jnp.bfloat16),
        mesh=mesh,
        scratch_types=[
            pltpu.VMEM((RPW * 8, H), jnp.bfloat16),
            pltpu.VMEM((NW * 8, H), jnp.bfloat16),
            pltpu.VMEM((RPW,), jnp.int32),
            pltpu.VMEM((RPW,), jnp.int32),
            pltpu.VMEM((RPW,), jnp.int32),
            pltpu.VMEM((RPW * 2, H), jnp.bfloat16),
            pltpu.SemaphoreType.DMA,
        ],
    )
    return f(o1, side, code, wlo, whi)


def _final_body(o2_ref, out_ref):
    x = o2_ref[...].astype(jnp.float32)
    x = x.reshape(B, 2, H)
    out_ref[...] = jnp.max(x, axis=1)


def _finalize(o2):
    return pl.pallas_call(
        _final_body,
        out_shape=jax.ShapeDtypeStruct((B, H), jnp.float32),
    )(o2)


@jax.jit
def _run(x, pos, batch, W1, b1, W2, b2):
    w1x = W1[:D].astype(jnp.bfloat16)
    w1p = W1[D:].astype(jnp.bfloat16)
    b1r = b1.reshape(1, H)
    b2r = b2.reshape(1, H)
    w2 = W2.astype(jnp.bfloat16)

    h = _mlp(x, pos, w1x, w1p, b1r, w2, b2r)

    ids = batch.astype(jnp.int32)
    cw = jnp.arange(NW, dtype=jnp.int32) * CROWS
    fids = ids[cw]
    lids = ids[cw + CROWS - 1]
    s = jnp.arange(B, dtype=jnp.int32)
    directly = jnp.any((fids[None, :] < s[:, None]) & (s[:, None] <= lids[None, :]),
                       axis=1)
    wlo = jnp.searchsorted(fids, s, side="left").astype(jnp.int32)
    whi = jnp.searchsorted(fids, s, side="right").astype(jnp.int32)
    nonempty = directly | (wlo < whi)
    code = jnp.where(nonempty, jnp.where(directly, 1, 2), 0).astype(jnp.int32)

    bias = jnp.zeros((8, H), jnp.bfloat16)
    bias = bias.at[1, :].set(jnp.bfloat16(_NEG)).at[2, :].set(jnp.bfloat16(_NEG))

    o1, side = _seg_phase1(h, ids, bias)
    o2 = _seg_phase2(o1, side, code, wlo, whi)
    out = _finalize(o2)

    pos_out = jnp.zeros((B, 3), dtype=pos.dtype)
    batch_out = jnp.arange(B, dtype=batch.dtype)
    return (out, pos_out, batch_out)


def kernel(x, pos, batch, W1, b1, W2, b2):
    return _run(x, pos, batch, W1, b1, W2, b2)


# speedup vs baseline: 1.3969x; 1.0053x over previous
import functools

import jax
import jax.numpy as jnp
import numpy as np
from jax import lax
from jax.experimental import pallas as pl
from jax.experimental.pallas import tpu as pltpu
from jax.experimental.pallas import tpu_sc as plsc

N = 320000
D = 128
H = 128
B = 1024

NW = 32
CROWS = N // NW
T = 400
NT = CROWS // T
KC = 8

MLP_R = 16000

_NEG = float(jnp.finfo(jnp.bfloat16).min)


def _mlp_body(x_ref, pos_ref, w1x_ref, w1p_ref, b1_ref, w2_ref, b2_ref, o_ref):
    xb = x_ref[...].astype(jnp.bfloat16)
    pb = pos_ref[...].astype(jnp.bfloat16)
    h = jnp.dot(xb, w1x_ref[...], preferred_element_type=jnp.float32)
    h += jnp.dot(pb, w1p_ref[...], preferred_element_type=jnp.float32)
    h = jnp.maximum(h + b1_ref[...], 0.0).astype(jnp.bfloat16)
    h = jnp.dot(h, w2_ref[...], preferred_element_type=jnp.float32)
    o_ref[...] = (h + b2_ref[...]).astype(jnp.bfloat16)


def _mlp(x, pos, w1x, w1p, b1, w2, b2):
    grid = (N // MLP_R,)
    return pl.pallas_call(
        _mlp_body,
        grid=grid,
        in_specs=[
            pl.BlockSpec((MLP_R, D), lambda k: (k, 0)),
            pl.BlockSpec((MLP_R, 3), lambda k: (k, 0)),
            pl.BlockSpec((D, H), lambda k: (0, 0)),
            pl.BlockSpec((3, H), lambda k: (0, 0)),
            pl.BlockSpec((1, H), lambda k: (0, 0)),
            pl.BlockSpec((H, H), lambda k: (0, 0)),
            pl.BlockSpec((1, H), lambda k: (0, 0)),
        ],
        out_specs=pl.BlockSpec((MLP_R, H), lambda k: (k, 0)),
        out_shape=jax.ShapeDtypeStruct((N, H), jnp.bfloat16),
    )(x, pos, w1x, w1p, b1, w2, b2)


def _seg_phase1_body(h_hbm, ids_hbm, bias_hbm, o1_hbm, side_hbm,
                     data_v, ids_v, m_buf, bias_v, sem):
    cid = lax.axis_index("c")
    sid = lax.axis_index("s")
    wid = sid * 2 + cid
    base = wid * CROWS

    neg2 = jnp.full((2, 16), _NEG, jnp.bfloat16)
    pltpu.sync_copy(bias_hbm, bias_v)

    def flush(pid, fid):
        def to_side():
            pltpu.sync_copy(m_buf, side_hbm.at[pl.ds(wid * 8, 8)])

        def to_out():
            pltpu.sync_copy(m_buf, o1_hbm.at[pl.ds(pid * 8, 8)])

        lax.cond(pid == fid, to_side, to_out)

    def tile_loop(t, carry):
        r0 = base + t * T
        pltpu.sync_copy(h_hbm.at[pl.ds(r0, T)], data_v)
        pltpu.sync_copy(ids_hbm.at[pl.ds(r0, T)], ids_v)

        def group_loop(q, gcarry):
            prev, fid = gcarry
            rbase = pl.multiple_of(q * 16, 16)
            ids16 = ids_v[pl.ds(rbase, 16)]
            i_first = ids16[0]
            i_last = ids16[15]
            uniform = (i_first == prev) & (i_last == prev)

            def fast_group():
                for c in range(KC):
                    acc = m_buf[pl.ds(0, 2), pl.ds(c * 16, 16)]
                    for u in range(8):
                        d = data_v[pl.ds(rbase + 2 * u, 2), pl.ds(c * 16, 16)]
                        acc = jnp.maximum(acc, d)
                    m_buf[pl.ds(0, 2), pl.ds(c * 16, 16)] = acc

            def slow_group():
                prev2, fid2 = prev, fid
                for u in range(8):
                    i0 = ids16[2 * u]
                    i1 = ids16[2 * u + 1]
                    first = fid2 < 0
                    fid2 = jnp.where(first, i0, fid2)
                    prev2 = jnp.where(first, i0, prev2)
                    flush0 = i0 != prev2
                    flush1 = i1 != i0

                    def do_flush0(pid=prev2, f=fid2):
                        flush(pid, f)

                    lax.cond(flush0, do_flush0, lambda: None)
                    mids = []
                    for c in range(KC):
                        d = data_v[pl.ds(rbase + 2 * u, 2), pl.ds(c * 16, 16)]
                        mold = m_buf[pl.ds(0, 2), pl.ds(c * 16, 16)]
                        b0 = bias_v[pl.ds(0, 2), pl.ds(c * 16, 16)]
                        m_mid = jnp.maximum(jnp.where(flush0, neg2, mold),
                                            d + b0)
                        m_buf[pl.ds(0, 2), pl.ds(c * 16, 16)] = m_mid
                        mids.append((m_mid, d))

                    def do_flush1(pid=i0, f=fid2):
                        flush(pid, f)

                    lax.cond(flush1, do_flush1, lambda: None)
                    for c in range(KC):
                        m_mid, d = mids[c]
                        b1 = bias_v[pl.ds(2, 2), pl.ds(c * 16, 16)]
                        m_new = jnp.maximum(jnp.where(flush1, neg2, m_mid),
                                            d + b1)
                        m_buf[pl.ds(0, 2), pl.ds(c * 16, 16)] = m_new
                    prev2 = i1

            lax.cond(uniform, fast_group, slow_group)
            new_fid = jnp.where(fid < 0, i_first, fid)
            return (i_last, new_fid)

        return lax.fori_loop(0, T // 16, group_loop, carry)

    for c in range(KC):
        m_buf[pl.ds(0, 2), pl.ds(c * 16, 16)] = neg2
    prev, fid = lax.fori_loop(0, NT, tile_loop,
                              (jnp.int32(-1), jnp.int32(-1)))
    flush(prev, fid)


def _seg_phase1(h, ids, bias):
    mesh = plsc.VectorSubcoreMesh(core_axis_name="c", subcore_axis_name="s")
    f = pl.kernel(
        _seg_phase1_body,
        out_type=[
            jax.ShapeDtypeStruct((B * 8, H), jnp.bfloat16),
            jax.ShapeDtypeStruct((NW * 8, H), jnp.bfloat16),
        ],
        mesh=mesh,
        scratch_types=[
            pltpu.VMEM((T, H), jnp.bfloat16),
            pltpu.VMEM((T,), jnp.int32),
            pltpu.VMEM((8, H), jnp.bfloat16),
            pltpu.VMEM((8, H), jnp.bfloat16),
            pltpu.SemaphoreType.DMA,
        ],
    )
    return f(h, ids, bias)


RPW = B // NW


def _seg_phase2_body(o1_hbm, side_hbm, code_hbm, wlo_hbm, whi_hbm, o2_hbm,
                     o1_v, side_v, code_v, wlo_v, whi_v, out_v, sem):
    cid = lax.axis_index("c")
    sid = lax.axis_index("s")
    wid = sid * 2 + cid
    base = wid * RPW

    neg2 = jnp.full((2, 16), _NEG, jnp.bfloat16)
    zero2 = jnp.zeros((2, 16), jnp.bfloat16)

    pltpu.sync_copy(o1_hbm.at[pl.ds(base * 8, RPW * 8)], o1_v)
    pltpu.sync_copy(side_hbm, side_v)
    pltpu.sync_copy(code_hbm.at[pl.ds(base, RPW)], code_v)
    pltpu.sync_copy(wlo_hbm.at[pl.ds(base, RPW)], wlo_v)
    pltpu.sync_copy(whi_hbm.at[pl.ds(base, RPW)], whi_v)

    for q in range(RPW // 16):
        code16 = code_v[pl.ds(q * 16, 16)]
        wlo16 = wlo_v[pl.ds(q * 16, 16)]
        whi16 = whi_v[pl.ds(q * 16, 16)]
        for j in range(16):
            row = q * 16 + j
            c = code16[j]
            lo = wlo16[j]
            hi = whi16[j]
            val = []
            for k in range(KC):
                o1k = o1_v[pl.ds(row * 8, 2), pl.ds(k * 16, 16)]
                v = jnp.where(c == 1, o1k, jnp.where(c == 0, zero2, neg2))
                val.append(v)

            def side_loop(w, vcarry):
                w8 = pl.multiple_of(w * 8, 8)
                return tuple(
                    jnp.maximum(vcarry[k],
                                side_v[pl.ds(w8, 2), pl.ds(k * 16, 16)])
                    for k in range(KC)
                )

            val = lax.fori_loop(lo, hi, side_loop, tuple(val))
            for k in range(KC):
                out_v[pl.ds(row * 2, 2), pl.ds(k * 16, 16)] = val[k]

    pltpu.sync_copy(out_v, o2_hbm.at[pl.ds(base * 2, RPW * 2)])


def _seg_phase2(o1, side, code, wlo, whi):
    mesh = plsc.VectorSubcoreMesh(core_axis_name="c", subcore_axis_name="s")
    f = pl.kernel(
        _seg_phase2_body,
        out_type=jax.ShapeDtypeStruct((B * 2, H), jnp.bfloat16),
        mesh=mesh,
        scratch_types=[
            pltpu.VMEM((RPW * 8, H), jnp.bfloat16),
            pltpu.VMEM((NW * 8, H), jnp.bfloat16),
            pltpu.VMEM((RPW,), jnp.int32),
            pltpu.VMEM((RPW,), jnp.int32),
            pltpu.VMEM((RPW,), jnp.int32),
            pltpu.VMEM((RPW * 2, H), jnp.bfloat16),
            pltpu.SemaphoreType.DMA,
        ],
    )
    return f(o1, side, code, wlo, whi)


def _final_body(o2_ref, out_ref):
    x = o2_ref[...].astype(jnp.float32)
    x = x.reshape(B, 2, H)
    out_ref[...] = jnp.max(x, axis=1)


def _finalize(o2):
    return pl.pallas_call(
        _final_body,
        out_shape=jax.ShapeDtypeStruct((B, H), jnp.float32),
    )(o2)


@jax.jit
def _run(x, pos, batch, W1, b1, W2, b2):
    w1x = W1[:D].astype(jnp.bfloat16)
    w1p = W1[D:].astype(jnp.bfloat16)
    b1r = b1.reshape(1, H)
    b2r = b2.reshape(1, H)
    w2 = W2.astype(jnp.bfloat16)

    h = _mlp(x, pos, w1x, w1p, b1r, w2, b2r)

    ids = batch.astype(jnp.int32)
    cw = jnp.arange(NW, dtype=jnp.int32) * CROWS
    fids = ids[cw]
    lids = ids[cw + CROWS - 1]
    s = jnp.arange(B, dtype=jnp.int32)
    directly = jnp.any((fids[None, :] < s[:, None]) & (s[:, None] <= lids[None, :]),
                       axis=1)
    wlo = jnp.searchsorted(fids, s, side="left").astype(jnp.int32)
    whi = jnp.searchsorted(fids, s, side="right").astype(jnp.int32)
    nonempty = directly | (wlo < whi)
    code = jnp.where(nonempty, jnp.where(directly, 1, 2), 0).astype(jnp.int32)

    bias = jnp.zeros((8, H), jnp.bfloat16)
    bias = bias.at[1, :].set(jnp.bfloat16(_NEG)).at[2, :].set(jnp.bfloat16(_NEG))

    o1, side = _seg_phase1(h, ids, bias)
    o2 = _seg_phase2(o1, side, code, wlo, whi)
    out = _finalize(o2)

    pos_out = jnp.zeros((B, 3), dtype=pos.dtype)
    batch_out = jnp.arange(B, dtype=batch.dtype)
    return (out, pos_out, batch_out)


def kernel(x, pos, batch, W1, b1, W2, b2):
    return _run(x, pos, batch, W1, b1, W2, b2)
